# baseline (speedup 1.0000x reference)
import functools

import jax
import jax.numpy as jnp
from jax import lax
from jax.experimental import pallas as pl
from jax.experimental.pallas import tpu as pltpu
from jax.experimental.pallas import tpu_sc as plsc

N_NODES = 10000
N_EDGES = 320000
D = 128
ALPHA = 0.5

NC, NS = 2, 16
NPAD = 10240
ROWS_PER_SUB = NPAD // NS
IDX_ROWS_P = 2560
E_PAD = IDX_ROWS_P * 128
NLOC = IDX_ROWS_P // NS
GRP = 16
NGRP = NLOC // GRP


def _sc_mesh():
    return plsc.VectorSubcoreMesh(core_axis_name="c", subcore_axis_name="s")


def _degree_kernel(sidx_r, zvec):
    @functools.partial(
        pl.kernel,
        out_type=jax.ShapeDtypeStruct((NC, NPAD), jnp.float32),
        mesh=_sc_mesh(),
        scratch_types=[
            pltpu.VMEM((3, GRP, 128), jnp.int32),
            pltpu.VMEM((128,), jnp.float32),
            pltpu.VMEM_SHARED((NPAD,), jnp.float32),
            pltpu.SemaphoreType.DMA((3,)),
            pltpu.SemaphoreType.DMA((2,)),
        ],
    )
    def k(sidx_hbm, zvec_hbm, deg_hbm, si_v, ones_v, acc, isem, ssem):
        c = lax.axis_index("c")
        s = lax.axis_index("s")
        r0 = s * NLOC

        @pl.loop(0, 128, step=16)
        def _(i):
            ones_v[pl.ds(i, 16)] = jnp.full((16,), 1.0, jnp.float32)

        for g in range(2):
            pltpu.async_copy(sidx_hbm.at[c, pl.ds(r0 + g * GRP, GRP)],
                             si_v.at[g], isem.at[g])

        pltpu.sync_copy(zvec_hbm, acc.at[pl.ds(s * ROWS_PER_SUB, ROWS_PER_SUB)])
        plsc.subcore_barrier()

        def wait_idx(bb):
            pltpu.make_async_copy(sidx_hbm.at[c, pl.ds(r0, GRP)],
                                  si_v.at[bb], isem.at[bb]).wait()

        def wait_sa(bx):
            pltpu.make_async_copy(ones_v, acc.at[si_v.at[0, 0]],
                                  ssem.at[bx]).wait()

        @pl.loop(0, NLOC, step=2)
        def _(k0):
            for ph in range(2):
                kk = k0 + ph
                g = kk // GRP
                j = lax.rem(kk, GRP)
                bg = lax.rem(g, 3)

                @pl.when(kk >= 2)
                def _():
                    wait_sa(ph)

                @pl.when(j == 0)
                def _():
                    for bb in range(3):
                        @pl.when(bg == bb)
                        def _():
                            wait_idx(bb)

                @pl.when(j == 1)
                def _():
                    for bb in range(3):
                        @pl.when((bg == bb) & (g + 2 < NGRP))
                        def _():
                            bn = (bb + 2) % 3
                            pltpu.async_copy(
                                sidx_hbm.at[c, pl.ds(r0 + (g + 2) * GRP, GRP)],
                                si_v.at[bn], isem.at[bn])

                pltpu.async_copy(ones_v, acc.at[si_v.at[bg, j]], ssem.at[ph],
                                 add=True)

        wait_sa(0)
        wait_sa(1)
        plsc.subcore_barrier()
        sl = pl.ds(s * ROWS_PER_SUB, ROWS_PER_SUB)
        pltpu.sync_copy(acc.at[sl], deg_hbm.at[c, sl])

    return k(sidx_r, zvec)


def _aggregate_kernel(zcat, gidx_r, sidx_r, zrows):
    @functools.partial(
        pl.kernel,
        out_type=jax.ShapeDtypeStruct((NC, NPAD, D), jnp.float32),
        mesh=_sc_mesh(),
        scratch_types=[
            pltpu.VMEM((3, GRP, 128), jnp.int32),
            pltpu.VMEM((3, GRP, 128), jnp.int32),
            pltpu.VMEM((2, 128, D), jnp.float32),
            pltpu.VMEM_SHARED((NPAD, D), jnp.float32),
            pltpu.SemaphoreType.DMA((3,)),
            pltpu.SemaphoreType.DMA((3,)),
            pltpu.SemaphoreType.DMA((2,)),
            pltpu.SemaphoreType.DMA((2,)),
        ],
    )
    def k(zcat_hbm, gidx_hbm, sidx_hbm, zrows_hbm, s_hbm,
          gi_v, si_v, gbuf, acc, isem_g, isem_s, gsem, ssem):
        c = lax.axis_index("c")
        s = lax.axis_index("s")
        r0 = s * NLOC

        for g in range(2):
            pltpu.async_copy(gidx_hbm.at[c, pl.ds(r0 + g * GRP, GRP)],
                             gi_v.at[g], isem_g.at[g])
            pltpu.async_copy(sidx_hbm.at[c, pl.ds(r0 + g * GRP, GRP)],
                             si_v.at[g], isem_s.at[g])

        pltpu.sync_copy(zrows_hbm, acc.at[pl.ds(s * ROWS_PER_SUB, ROWS_PER_SUB)])
        plsc.subcore_barrier()

        def wait_idx(bb):
            pltpu.make_async_copy(gidx_hbm.at[c, pl.ds(r0, GRP)],
                                  gi_v.at[bb], isem_g.at[bb]).wait()
            pltpu.make_async_copy(sidx_hbm.at[c, pl.ds(r0, GRP)],
                                  si_v.at[bb], isem_s.at[bb]).wait()

        def wait_gather(bx):
            pltpu.make_async_copy(zcat_hbm.at[gi_v.at[0, 0]], gbuf.at[bx],
                                  gsem.at[bx]).wait()

        def wait_sa(bx):
            pltpu.make_async_copy(gbuf.at[bx], acc.at[si_v.at[0, 0]],
                                  ssem.at[bx]).wait()

        def issue_sa(kk, bx):
            g = kk // GRP
            j = lax.rem(kk, GRP)
            bg = lax.rem(g, 3)
            pltpu.async_copy(gbuf.at[bx], acc.at[si_v.at[bg, j]],
                             ssem.at[bx], add=True)

        @pl.loop(0, NLOC, step=2)
        def _(k0):
            for ph in range(2):
                kk = k0 + ph
                g = kk // GRP
                j = lax.rem(kk, GRP)
                bg = lax.rem(g, 3)

                @pl.when(kk >= 2)
                def _():
                    wait_sa(ph)

                @pl.when(j == 0)
                def _():
                    for bb in range(3):
                        @pl.when(bg == bb)
                        def _():
                            wait_idx(bb)

                @pl.when(j == 1)
                def _():
                    for bb in range(3):
                        @pl.when((bg == bb) & (g + 2 < NGRP))
                        def _():
                            bn = (bb + 2) % 3
                            pltpu.async_copy(
                                gidx_hbm.at[c, pl.ds(r0 + (g + 2) * GRP, GRP)],
                                gi_v.at[bn], isem_g.at[bn])
                            pltpu.async_copy(
                                sidx_hbm.at[c, pl.ds(r0 + (g + 2) * GRP, GRP)],
                                si_v.at[bn], isem_s.at[bn])

                pltpu.async_copy(zcat_hbm.at[gi_v.at[bg, j]], gbuf.at[ph],
                                 gsem.at[ph])

                @pl.when(kk >= 1)
                def _():
                    wait_gather(1 - ph)
                    issue_sa(kk - 1, 1 - ph)

        wait_gather(1)
        issue_sa(NLOC - 1, 1)
        wait_sa(0)
        wait_sa(1)
        plsc.subcore_barrier()
        sl = pl.ds(s * ROWS_PER_SUB, ROWS_PER_SUB)
        pltpu.sync_copy(acc.at[sl], s_hbm.at[c, sl])

    return k(zcat, gidx_r, sidx_r, zrows)


_BLK = 5120
_NBLK = NPAD // _BLK


def _inv_sqrt_cols(deg_blk):
    degt = jnp.transpose(deg_blk)
    oinv = jnp.where(degt[:, 0:1] > 0, lax.rsqrt(degt[:, 0:1]), 0.0)
    iinv = jnp.where(degt[:, 1:2] > 0, lax.rsqrt(degt[:, 1:2]), 0.0)
    return oinv, iinv


def _transform_body(x_ref, w_ref, deg_ref, z_ref):
    y = jnp.dot(x_ref[...], w_ref[...], precision=lax.Precision.HIGHEST,
                preferred_element_type=jnp.float32)
    oinv, iinv = _inv_sqrt_cols(deg_ref[...])
    z_ref[0] = iinv * y[:, :D]
    z_ref[1] = oinv * y[:, D:]


def _transform(x, wcat, deg):
    return pl.pallas_call(
        _transform_body,
        grid=(_NBLK,),
        in_specs=[
            pl.BlockSpec((_BLK, D), lambda i: (i, 0)),
            pl.BlockSpec((D, 2 * D), lambda i: (0, 0)),
            pl.BlockSpec((2, _BLK), lambda i: (0, i)),
        ],
        out_specs=pl.BlockSpec((2, _BLK, D), lambda i: (0, i, 0)),
        out_shape=jax.ShapeDtypeStruct((2, NPAD, D), jnp.float32),
    )(x, wcat, deg)


def _combine_body(s_ref, deg_ref, b_ref, o_ref):
    oinv, iinv = _inv_sqrt_cols(deg_ref[...])
    o_ref[...] = oinv * s_ref[0] + iinv * s_ref[1] + b_ref[...]


def _combine(S, deg, bias):
    return pl.pallas_call(
        _combine_body,
        grid=(_NBLK,),
        in_specs=[
            pl.BlockSpec((2, _BLK, D), lambda i: (0, i, 0)),
            pl.BlockSpec((2, _BLK), lambda i: (0, i)),
            pl.BlockSpec((1, D), lambda i: (0, 0)),
        ],
        out_specs=pl.BlockSpec((_BLK, D), lambda i: (i, 0)),
        out_shape=jax.ShapeDtypeStruct((N_NODES, D), jnp.float32),
    )(S, deg, bias)


def kernel(x, edge_index, W_src, b_src, W_dst, b_dst):
    pad_n = E_PAD - N_EDGES
    j = jnp.arange(pad_n, dtype=jnp.int32)
    pad_s = N_NODES + jnp.remainder(j, NPAD - N_NODES)
    pad_g = jnp.remainder(j, N_NODES)
    pad_s2 = jnp.stack([pad_s, pad_s])
    sidx_r = jnp.concatenate([edge_index, pad_s2], axis=1) \
                .reshape(NC, IDX_ROWS_P, 128)
    g0 = jnp.concatenate([edge_index[1], pad_g])
    g1 = jnp.concatenate([edge_index[0], pad_g]) + NPAD
    gidx = jnp.stack([g0, g1]).reshape(NC, IDX_ROWS_P, 128)

    zvec = jnp.zeros((ROWS_PER_SUB,), jnp.float32)
    zrows = jnp.zeros((ROWS_PER_SUB, D), jnp.float32)

    deg = _degree_kernel(sidx_r, zvec)

    wcat = jnp.concatenate([ALPHA * W_src, (1.0 - ALPHA) * W_dst], axis=1)
    zcat = _transform(x, wcat, deg).reshape(2 * NPAD, D)

    S = _aggregate_kernel(zcat, gidx, sidx_r, zrows)

    bias = (ALPHA * b_src + (1.0 - ALPHA) * b_dst).reshape(1, D)
    return _combine(S, deg, bias)


# speedup vs baseline: 1.0094x; 1.0094x over previous
import functools

import jax
import jax.numpy as jnp
from jax import lax
from jax.experimental import pallas as pl
from jax.experimental.pallas import tpu as pltpu
from jax.experimental.pallas import tpu_sc as plsc

N_NODES = 10000
N_EDGES = 320000
D = 128
ALPHA = 0.5

NC, NS = 2, 16
NPAD = 10240
ROWS_PER_SUB = NPAD // NS
IDX_ROWS_P = 2560
E_PAD = IDX_ROWS_P * 128
NLOC = IDX_ROWS_P // NS
GRP = 16
NGRP = NLOC // GRP


def _sc_mesh():
    return plsc.VectorSubcoreMesh(core_axis_name="c", subcore_axis_name="s")


def _degree_kernel(sidx_r, zvec):
    @functools.partial(
        pl.kernel,
        out_type=jax.ShapeDtypeStruct((NC, NPAD), jnp.float32),
        mesh=_sc_mesh(),
        scratch_types=[
            pltpu.VMEM((3, GRP, 128), jnp.int32),
            pltpu.VMEM((128,), jnp.float32),
            pltpu.VMEM_SHARED((NPAD,), jnp.float32),
            pltpu.SemaphoreType.DMA((3,)),
            pltpu.SemaphoreType.DMA((2,)),
        ],
    )
    def k(sidx_hbm, zvec_hbm, deg_hbm, si_v, ones_v, acc, isem, ssem):
        c = lax.axis_index("c")
        s = lax.axis_index("s")
        r0 = s * NLOC

        @pl.loop(0, 128, step=16)
        def _(i):
            ones_v[pl.ds(i, 16)] = jnp.full((16,), 1.0, jnp.float32)

        for g in range(2):
            pltpu.async_copy(sidx_hbm.at[c, pl.ds(r0 + g * GRP, GRP)],
                             si_v.at[g], isem.at[g])

        pltpu.sync_copy(zvec_hbm, acc.at[pl.ds(s * ROWS_PER_SUB, ROWS_PER_SUB)])
        plsc.subcore_barrier()

        def wait_idx(bb):
            pltpu.make_async_copy(sidx_hbm.at[c, pl.ds(r0, GRP)],
                                  si_v.at[bb], isem.at[bb]).wait()

        def wait_sa(bx):
            pltpu.make_async_copy(ones_v, acc.at[si_v.at[0, 0]],
                                  ssem.at[bx]).wait()

        @pl.loop(0, NLOC, step=2)
        def _(k0):
            for ph in range(2):
                kk = k0 + ph
                g = kk // GRP
                j = lax.rem(kk, GRP)
                bg = lax.rem(g, 3)

                @pl.when(kk >= 2)
                def _():
                    wait_sa(ph)

                @pl.when(j == 0)
                def _():
                    for bb in range(3):
                        @pl.when(bg == bb)
                        def _():
                            wait_idx(bb)

                @pl.when(j == 1)
                def _():
                    for bb in range(3):
                        @pl.when((bg == bb) & (g + 2 < NGRP))
                        def _():
                            bn = (bb + 2) % 3
                            pltpu.async_copy(
                                sidx_hbm.at[c, pl.ds(r0 + (g + 2) * GRP, GRP)],
                                si_v.at[bn], isem.at[bn])

                pltpu.async_copy(ones_v, acc.at[si_v.at[bg, j]], ssem.at[ph],
                                 add=True)

        wait_sa(0)
        wait_sa(1)
        plsc.subcore_barrier()
        sl = pl.ds(s * ROWS_PER_SUB, ROWS_PER_SUB)
        pltpu.sync_copy(acc.at[sl], deg_hbm.at[c, sl])

    return k(sidx_r, zvec)


def _aggregate_kernel(zcat, gidx_r, sidx_r, zrows):
    @functools.partial(
        pl.kernel,
        out_type=jax.ShapeDtypeStruct((NC, NPAD, D), jnp.float32),
        mesh=_sc_mesh(),
        scratch_types=[
            pltpu.VMEM((3, GRP, 128), jnp.int32),
            pltpu.VMEM((3, GRP, 128), jnp.int32),
            pltpu.VMEM((2, 128, D), jnp.float32),
            pltpu.VMEM_SHARED((NPAD, D), jnp.float32),
            pltpu.SemaphoreType.DMA((3,)),
            pltpu.SemaphoreType.DMA((3,)),
            pltpu.SemaphoreType.DMA((2,)),
            pltpu.SemaphoreType.DMA((2,)),
        ],
    )
    def k(zcat_hbm, gidx_hbm, sidx_hbm, zrows_hbm, s_hbm,
          gi_v, si_v, gbuf, acc, isem_g, isem_s, gsem, ssem):
        c = lax.axis_index("c")
        s = lax.axis_index("s")
        r0 = s * NLOC

        for g in range(2):
            pltpu.async_copy(gidx_hbm.at[c, pl.ds(r0 + g * GRP, GRP)],
                             gi_v.at[g], isem_g.at[g])
            pltpu.async_copy(sidx_hbm.at[c, pl.ds(r0 + g * GRP, GRP)],
                             si_v.at[g], isem_s.at[g])

        pltpu.sync_copy(zrows_hbm, acc.at[pl.ds(s * ROWS_PER_SUB, ROWS_PER_SUB)])
        plsc.subcore_barrier()

        def wait_idx(bb):
            pltpu.make_async_copy(gidx_hbm.at[c, pl.ds(r0, GRP)],
                                  gi_v.at[bb], isem_g.at[bb]).wait()
            pltpu.make_async_copy(sidx_hbm.at[c, pl.ds(r0, GRP)],
                                  si_v.at[bb], isem_s.at[bb]).wait()

        def wait_gather(bx):
            pltpu.make_async_copy(zcat_hbm.at[gi_v.at[0, 0]], gbuf.at[bx],
                                  gsem.at[bx]).wait()

        def wait_sa(bx):
            pltpu.make_async_copy(gbuf.at[bx], acc.at[si_v.at[0, 0]],
                                  ssem.at[bx]).wait()

        def issue_sa(kk, bx):
            g = kk // GRP
            j = lax.rem(kk, GRP)
            bg = lax.rem(g, 3)
            pltpu.async_copy(gbuf.at[bx], acc.at[si_v.at[bg, j]],
                             ssem.at[bx], add=True)

        @pl.loop(0, NLOC, step=2)
        def _(k0):
            for ph in range(2):
                kk = k0 + ph
                g = kk // GRP
                j = lax.rem(kk, GRP)
                bg = lax.rem(g, 3)

                @pl.when(kk >= 2)
                def _():
                    wait_sa(ph)

                @pl.when(j == 0)
                def _():
                    for bb in range(3):
                        @pl.when(bg == bb)
                        def _():
                            wait_idx(bb)

                @pl.when(j == 1)
                def _():
                    for bb in range(3):
                        @pl.when((bg == bb) & (g + 2 < NGRP))
                        def _():
                            bn = (bb + 2) % 3
                            pltpu.async_copy(
                                gidx_hbm.at[c, pl.ds(r0 + (g + 2) * GRP, GRP)],
                                gi_v.at[bn], isem_g.at[bn])
                            pltpu.async_copy(
                                sidx_hbm.at[c, pl.ds(r0 + (g + 2) * GRP, GRP)],
                                si_v.at[bn], isem_s.at[bn])

                pltpu.async_copy(zcat_hbm.at[gi_v.at[bg, j]], gbuf.at[ph],
                                 gsem.at[ph])

                @pl.when(kk >= 1)
                def _():
                    wait_gather(1 - ph)
                    issue_sa(kk - 1, 1 - ph)

        wait_gather(1)
        issue_sa(NLOC - 1, 1)
        wait_sa(0)
        wait_sa(1)
        plsc.subcore_barrier()
        sl = pl.ds(s * ROWS_PER_SUB, ROWS_PER_SUB)
        pltpu.sync_copy(acc.at[sl], s_hbm.at[c, sl])

    return k(zcat, gidx_r, sidx_r, zrows)


_BLK = 2560
_NBLK = NPAD // _BLK


def _inv_sqrt_cols(deg_blk):
    degt = jnp.transpose(deg_blk)
    oinv = jnp.where(degt[:, 0:1] > 0, lax.rsqrt(degt[:, 0:1]), 0.0)
    iinv = jnp.where(degt[:, 1:2] > 0, lax.rsqrt(degt[:, 1:2]), 0.0)
    return oinv, iinv


def _transform_body(x_ref, w_ref, deg_ref, z_ref):
    y = jnp.dot(x_ref[...], w_ref[...], precision=lax.Precision.HIGHEST,
                preferred_element_type=jnp.float32)
    oinv, iinv = _inv_sqrt_cols(deg_ref[...])
    z_ref[0] = iinv * y[:, :D]
    z_ref[1] = oinv * y[:, D:]


def _transform(x, wcat, deg):
    return pl.pallas_call(
        _transform_body,
        grid=(_NBLK,),
        in_specs=[
            pl.BlockSpec((_BLK, D), lambda i: (i, 0)),
            pl.BlockSpec((D, 2 * D), lambda i: (0, 0)),
            pl.BlockSpec((2, _BLK), lambda i: (0, i)),
        ],
        out_specs=pl.BlockSpec((2, _BLK, D), lambda i: (0, i, 0)),
        out_shape=jax.ShapeDtypeStruct((2, NPAD, D), jnp.float32),
    )(x, wcat, deg)


def _combine_body(s_ref, deg_ref, b_ref, o_ref):
    oinv, iinv = _inv_sqrt_cols(deg_ref[...])
    o_ref[...] = oinv * s_ref[0] + iinv * s_ref[1] + b_ref[...]


def _combine(S, deg, bias):
    return pl.pallas_call(
        _combine_body,
        grid=(_NBLK,),
        in_specs=[
            pl.BlockSpec((2, _BLK, D), lambda i: (0, i, 0)),
            pl.BlockSpec((2, _BLK), lambda i: (0, i)),
            pl.BlockSpec((1, D), lambda i: (0, 0)),
        ],
        out_specs=pl.BlockSpec((_BLK, D), lambda i: (i, 0)),
        out_shape=jax.ShapeDtypeStruct((N_NODES, D), jnp.float32),
    )(S, deg, bias)


def kernel(x, edge_index, W_src, b_src, W_dst, b_dst):
    pad_n = E_PAD - N_EDGES
    j = jnp.arange(pad_n, dtype=jnp.int32)
    pad_s = N_NODES + jnp.remainder(j, NPAD - N_NODES)
    pad_g = jnp.remainder(j, N_NODES)
    pad_s2 = jnp.stack([pad_s, pad_s])
    sidx_r = jnp.concatenate([edge_index, pad_s2], axis=1) \
                .reshape(NC, IDX_ROWS_P, 128)
    g0 = jnp.concatenate([edge_index[1], pad_g])
    g1 = jnp.concatenate([edge_index[0], pad_g]) + NPAD
    gidx = jnp.stack([g0, g1]).reshape(NC, IDX_ROWS_P, 128)

    zvec = jnp.zeros((ROWS_PER_SUB,), jnp.float32)
    zrows = jnp.zeros((ROWS_PER_SUB, D), jnp.float32)

    deg = _degree_kernel(sidx_r, zvec)

    wcat = jnp.concatenate([ALPHA * W_src, (1.0 - ALPHA) * W_dst], axis=1)
    zcat = _transform(x, wcat, deg).reshape(2 * NPAD, D)

    S = _aggregate_kernel(zcat, gidx, sidx_r, zrows)

    bias = (ALPHA * b_src + (1.0 - ALPHA) * b_dst).reshape(1, D)
    return _combine(S, deg, bias)


# speedup vs baseline: 1.0286x; 1.0191x over previous
import functools

import jax
import jax.numpy as jnp
from jax import lax
from jax.experimental import pallas as pl
from jax.experimental.pallas import tpu as pltpu
from jax.experimental.pallas import tpu_sc as plsc

N_NODES = 10000
N_EDGES = 320000
D = 128
ALPHA = 0.5

NC, NS = 2, 16
NPAD = 10240
ROWS_PER_SUB = NPAD // NS
IDX_ROWS_P = 2560
E_PAD = IDX_ROWS_P * 128
NLOC = IDX_ROWS_P // NS
GRP = 16
NGRP = NLOC // GRP


def _sc_mesh():
    return plsc.VectorSubcoreMesh(core_axis_name="c", subcore_axis_name="s")


def _degree_kernel(sidx_r, zvec):
    @functools.partial(
        pl.kernel,
        out_type=jax.ShapeDtypeStruct((NC, NPAD), jnp.float32),
        mesh=_sc_mesh(),
        scratch_types=[
            pltpu.VMEM((3, GRP, 128), jnp.int32),
            pltpu.VMEM((128,), jnp.float32),
            pltpu.VMEM_SHARED((NPAD,), jnp.float32),
            pltpu.SemaphoreType.DMA((3,)),
            pltpu.SemaphoreType.DMA((2,)),
        ],
    )
    def k(sidx_hbm, zvec_hbm, deg_hbm, si_v, ones_v, acc, isem, ssem):
        c = lax.axis_index("c")
        s = lax.axis_index("s")
        r0 = s * NLOC

        @pl.loop(0, 128, step=16)
        def _(i):
            ones_v[pl.ds(i, 16)] = jnp.full((16,), 1.0, jnp.float32)

        for g in range(2):
            pltpu.async_copy(sidx_hbm.at[c, pl.ds(r0 + g * GRP, GRP)],
                             si_v.at[g], isem.at[g])

        pltpu.sync_copy(zvec_hbm, acc.at[pl.ds(s * ROWS_PER_SUB, ROWS_PER_SUB)])
        plsc.subcore_barrier()

        def wait_idx(bb):
            pltpu.make_async_copy(sidx_hbm.at[c, pl.ds(r0, GRP)],
                                  si_v.at[bb], isem.at[bb]).wait()

        def wait_sa(bx):
            pltpu.make_async_copy(ones_v, acc.at[si_v.at[0, 0]],
                                  ssem.at[bx]).wait()

        @pl.loop(0, NLOC, step=2)
        def _(k0):
            for ph in range(2):
                kk = k0 + ph
                g = kk // GRP
                j = lax.rem(kk, GRP)
                bg = lax.rem(g, 3)

                @pl.when(kk >= 2)
                def _():
                    wait_sa(ph)

                @pl.when(j == 0)
                def _():
                    for bb in range(3):
                        @pl.when(bg == bb)
                        def _():
                            wait_idx(bb)

                @pl.when(j == 1)
                def _():
                    for bb in range(3):
                        @pl.when((bg == bb) & (g + 2 < NGRP))
                        def _():
                            bn = (bb + 2) % 3
                            pltpu.async_copy(
                                sidx_hbm.at[c, pl.ds(r0 + (g + 2) * GRP, GRP)],
                                si_v.at[bn], isem.at[bn])

                pltpu.async_copy(ones_v, acc.at[si_v.at[bg, j]], ssem.at[ph],
                                 add=True)

        wait_sa(0)
        wait_sa(1)
        plsc.subcore_barrier()
        sl = pl.ds(s * ROWS_PER_SUB, ROWS_PER_SUB)
        pltpu.sync_copy(acc.at[sl], deg_hbm.at[c, sl])

    return k(sidx_r, zvec)


def _aggregate_kernel(zcat, gidx_r, sidx_r):
    @functools.partial(
        pl.kernel,
        out_type=jax.ShapeDtypeStruct((NC, NPAD, D), jnp.float32),
        mesh=_sc_mesh(),
        scratch_types=[
            pltpu.VMEM((3, GRP, 128), jnp.int32),
            pltpu.VMEM((3, GRP, 128), jnp.int32),
            pltpu.VMEM((2, 128, D), jnp.float32),
            pltpu.VMEM_SHARED((NPAD, D), jnp.float32),
            pltpu.SemaphoreType.DMA((3,)),
            pltpu.SemaphoreType.DMA((3,)),
            pltpu.SemaphoreType.DMA((2,)),
            pltpu.SemaphoreType.DMA((2,)),
        ],
    )
    def k(zcat_hbm, gidx_hbm, sidx_hbm, s_hbm,
          gi_v, si_v, gbuf, acc, isem_g, isem_s, gsem, ssem):
        c = lax.axis_index("c")
        s = lax.axis_index("s")
        r0 = s * NLOC

        for g in range(2):
            pltpu.async_copy(gidx_hbm.at[c, pl.ds(r0 + g * GRP, GRP)],
                             gi_v.at[g], isem_g.at[g])
            pltpu.async_copy(sidx_hbm.at[c, pl.ds(r0 + g * GRP, GRP)],
                             si_v.at[g], isem_s.at[g])

        @pl.loop(0, 32)
        def _(i):
            @pl.loop(0, D, step=16)
            def _(cc):
                gbuf[0, i, pl.ds(cc, 16)] = jnp.zeros((16,), jnp.float32)

        @pl.loop(0, ROWS_PER_SUB, step=32)
        def _(r):
            pltpu.async_copy(gbuf.at[0, pl.ds(0, 32)],
                             acc.at[pl.ds(s * ROWS_PER_SUB + r, 32)],
                             gsem.at[0])

        @pl.loop(0, ROWS_PER_SUB, step=32)
        def _(r):
            pltpu.make_async_copy(gbuf.at[0, pl.ds(0, 32)],
                                  acc.at[pl.ds(0, 32)], gsem.at[0]).wait()

        plsc.subcore_barrier()

        def wait_idx(bb):
            pltpu.make_async_copy(gidx_hbm.at[c, pl.ds(r0, GRP)],
                                  gi_v.at[bb], isem_g.at[bb]).wait()
            pltpu.make_async_copy(sidx_hbm.at[c, pl.ds(r0, GRP)],
                                  si_v.at[bb], isem_s.at[bb]).wait()

        def wait_gather(bx):
            pltpu.make_async_copy(zcat_hbm.at[gi_v.at[0, 0]], gbuf.at[bx],
                                  gsem.at[bx]).wait()

        def wait_sa(bx):
            pltpu.make_async_copy(gbuf.at[bx], acc.at[si_v.at[0, 0]],
                                  ssem.at[bx]).wait()

        def issue_sa(kk, bx):
            g = kk // GRP
            j = lax.rem(kk, GRP)
            bg = lax.rem(g, 3)
            pltpu.async_copy(gbuf.at[bx], acc.at[si_v.at[bg, j]],
                             ssem.at[bx], add=True)

        @pl.loop(0, NLOC, step=2)
        def _(k0):
            for ph in range(2):
                kk = k0 + ph
                g = kk // GRP
                j = lax.rem(kk, GRP)
                bg = lax.rem(g, 3)

                @pl.when(kk >= 2)
                def _():
                    wait_sa(ph)

                @pl.when(j == 0)
                def _():
                    for bb in range(3):
                        @pl.when(bg == bb)
                        def _():
                            wait_idx(bb)

                @pl.when(j == 1)
                def _():
                    for bb in range(3):
                        @pl.when((bg == bb) & (g + 2 < NGRP))
                        def _():
                            bn = (bb + 2) % 3
                            pltpu.async_copy(
                                gidx_hbm.at[c, pl.ds(r0 + (g + 2) * GRP, GRP)],
                                gi_v.at[bn], isem_g.at[bn])
                            pltpu.async_copy(
                                sidx_hbm.at[c, pl.ds(r0 + (g + 2) * GRP, GRP)],
                                si_v.at[bn], isem_s.at[bn])

                pltpu.async_copy(zcat_hbm.at[gi_v.at[bg, j]], gbuf.at[ph],
                                 gsem.at[ph])

                @pl.when(kk >= 1)
                def _():
                    wait_gather(1 - ph)
                    issue_sa(kk - 1, 1 - ph)

        wait_gather(1)
        issue_sa(NLOC - 1, 1)
        wait_sa(0)
        wait_sa(1)
        plsc.subcore_barrier()
        sl = pl.ds(s * ROWS_PER_SUB, ROWS_PER_SUB)
        pltpu.sync_copy(acc.at[sl], s_hbm.at[c, sl])

    return k(zcat, gidx_r, sidx_r)


_BLK = 2560
_NBLK = NPAD // _BLK


def _inv_sqrt_cols(deg_blk):
    degt = jnp.transpose(deg_blk)
    oinv = jnp.where(degt[:, 0:1] > 0, lax.rsqrt(degt[:, 0:1]), 0.0)
    iinv = jnp.where(degt[:, 1:2] > 0, lax.rsqrt(degt[:, 1:2]), 0.0)
    return oinv, iinv


def _transform_body(x_ref, w_ref, deg_ref, z_ref):
    y = jnp.dot(x_ref[...], w_ref[...], precision=lax.Precision.HIGHEST,
                preferred_element_type=jnp.float32)
    oinv, iinv = _inv_sqrt_cols(deg_ref[...])
    z_ref[0] = iinv * y[:, :D]
    z_ref[1] = oinv * y[:, D:]


def _transform(x, wcat, deg):
    return pl.pallas_call(
        _transform_body,
        grid=(_NBLK,),
        in_specs=[
            pl.BlockSpec((_BLK, D), lambda i: (i, 0)),
            pl.BlockSpec((D, 2 * D), lambda i: (0, 0)),
            pl.BlockSpec((2, _BLK), lambda i: (0, i)),
        ],
        out_specs=pl.BlockSpec((2, _BLK, D), lambda i: (0, i, 0)),
        out_shape=jax.ShapeDtypeStruct((2, NPAD, D), jnp.float32),
    )(x, wcat, deg)


def _combine_body(s_ref, deg_ref, b_ref, o_ref):
    oinv, iinv = _inv_sqrt_cols(deg_ref[...])
    o_ref[...] = oinv * s_ref[0] + iinv * s_ref[1] + b_ref[...]


def _combine(S, deg, bias):
    return pl.pallas_call(
        _combine_body,
        grid=(_NBLK,),
        in_specs=[
            pl.BlockSpec((2, _BLK, D), lambda i: (0, i, 0)),
            pl.BlockSpec((2, _BLK), lambda i: (0, i)),
            pl.BlockSpec((1, D), lambda i: (0, 0)),
        ],
        out_specs=pl.BlockSpec((_BLK, D), lambda i: (i, 0)),
        out_shape=jax.ShapeDtypeStruct((N_NODES, D), jnp.float32),
    )(S, deg, bias)


def kernel(x, edge_index, W_src, b_src, W_dst, b_dst):
    pad_n = E_PAD - N_EDGES
    j = jnp.arange(pad_n, dtype=jnp.int32)
    pad_s = N_NODES + jnp.remainder(j, NPAD - N_NODES)
    pad_g = jnp.remainder(j, N_NODES)
    pad_s2 = jnp.stack([pad_s, pad_s])
    sidx_r = jnp.concatenate([edge_index, pad_s2], axis=1) \
                .reshape(NC, IDX_ROWS_P, 128)
    g0 = jnp.concatenate([edge_index[1], pad_g])
    g1 = jnp.concatenate([edge_index[0], pad_g]) + NPAD
    gidx = jnp.stack([g0, g1]).reshape(NC, IDX_ROWS_P, 128)

    zvec = jnp.zeros((ROWS_PER_SUB,), jnp.float32)

    deg = _degree_kernel(sidx_r, zvec)

    wcat = jnp.concatenate([ALPHA * W_src, (1.0 - ALPHA) * W_dst], axis=1)
    zcat = _transform(x, wcat, deg).reshape(2 * NPAD, D)

    S = _aggregate_kernel(zcat, gidx, sidx_r)

    bias = (ALPHA * b_src + (1.0 - ALPHA) * b_dst).reshape(1, D)
    return _combine(S, deg, bias)
